# baseline (speedup 1.0000x reference)
import functools

import jax
import jax.numpy as jnp
from jax import lax
from jax.experimental import pallas as pl
from jax.experimental.pallas import tpu as pltpu
from jax.experimental.pallas import tpu_sc as plsc

N = 10000
E = 320000
D = 128
H = 128
T = 4
OUT = 64

NPAD = 10240
WA = 144
NB = 1024
NC = 2
NS = 16
NW = NC * NS
K = 64
CPW = (E + NW * K - 1) // (NW * K)
C0 = 137
C1 = 2 * CPW - C0
TOT = NS * (C0 + C1)
EPAD = TOT * K
GRID = NPAD // NB
ACCN = N
STRIPE = ACCN // NS


def _embed_body(x_ref, w_ref, a2_ref, haug_ref, sd_ref):
    h = lax.dot_general(x_ref[...], w_ref[...], (((1,), (0,)), ((), ())),
                        preferred_element_type=jnp.float32,
                        precision=lax.Precision.HIGHEST)
    ones_col = (lax.broadcasted_iota(jnp.int32, (NB, WA - D), 1) == 0)
    haug_ref[...] = jnp.concatenate([h, ones_col.astype(jnp.float32)], axis=1)
    sd_ref[...] = lax.dot_general(a2_ref[...], h, (((1,), (1,)), ((), ())),
                                  preferred_element_type=jnp.float32,
                                  precision=lax.Precision.HIGHEST)


def _embed(xp, W, A2):
    return pl.pallas_call(
        _embed_body,
        grid=(GRID,),
        in_specs=[
            pl.BlockSpec((NB, D), lambda i: (i, 0)),
            pl.BlockSpec((D, H), lambda i: (0, 0)),
            pl.BlockSpec((8, D), lambda i: (0, 0)),
        ],
        out_specs=[
            pl.BlockSpec((NB, WA), lambda i: (i, 0)),
            pl.BlockSpec((8, NB), lambda i: (0, i)),
        ],
        out_shape=[
            jax.ShapeDtypeStruct((NPAD, WA), jnp.float32),
            jax.ShapeDtypeStruct((8, NPAD), jnp.float32),
        ],
    )(xp, W, A2)


def _mid_body(parts_ref, w_ref, a2_ref, haug_ref, sd_ref):
    p = parts_ref[0] + parts_ref[1]
    g = jnp.maximum(p[:, :D] / (p[:, D:D + 1] + 1e-16), 0.0)
    h = lax.dot_general(g, w_ref[...], (((1,), (0,)), ((), ())),
                        preferred_element_type=jnp.float32,
                        precision=lax.Precision.HIGHEST)
    ones_col = (lax.broadcasted_iota(jnp.int32, (NB, WA - D), 1) == 0)
    haug_ref[...] = jnp.concatenate([h, ones_col.astype(jnp.float32)], axis=1)
    sd_ref[...] = lax.dot_general(a2_ref[...], h, (((1,), (1,)), ((), ())),
                                  preferred_element_type=jnp.float32,
                                  precision=lax.Precision.HIGHEST)


def _mid(parts, W, A2):
    return pl.pallas_call(
        _mid_body,
        grid=(GRID,),
        in_specs=[
            pl.BlockSpec((NC, NB, WA), lambda i: (0, i, 0)),
            pl.BlockSpec((H, H), lambda i: (0, 0)),
            pl.BlockSpec((8, H), lambda i: (0, 0)),
        ],
        out_specs=[
            pl.BlockSpec((NB, WA), lambda i: (i, 0)),
            pl.BlockSpec((8, NB), lambda i: (0, i)),
        ],
        out_shape=[
            jax.ShapeDtypeStruct((NPAD, WA), jnp.float32),
            jax.ShapeDtypeStruct((8, NPAD), jnp.float32),
        ],
    )(parts, W, A2)


def _head_body(parts_ref, l1w_ref, l1b_ref, fw_ref, fb_ref, out_ref):
    p = parts_ref[0] + parts_ref[1]
    g = jnp.maximum(p[:, :D] / (p[:, D:D + 1] + 1e-16), 0.0)
    t = lax.dot_general(g, l1w_ref[...], (((1,), (0,)), ((), ())),
                        preferred_element_type=jnp.float32,
                        precision=lax.Precision.HIGHEST)
    t = jnp.maximum(t + l1b_ref[...], 0.0)
    o = lax.dot_general(t, fw_ref[...], (((1,), (0,)), ((), ())),
                        preferred_element_type=jnp.float32,
                        precision=lax.Precision.HIGHEST)
    out_ref[...] = o + fb_ref[...]


def _head(parts, l1w, l1b, fw, fb):
    return pl.pallas_call(
        _head_body,
        grid=(GRID,),
        in_specs=[
            pl.BlockSpec((NC, NB, WA), lambda i: (0, i, 0)),
            pl.BlockSpec((H, H), lambda i: (0, 0)),
            pl.BlockSpec((1, H), lambda i: (0, 0)),
            pl.BlockSpec((H, OUT), lambda i: (0, 0)),
            pl.BlockSpec((1, OUT), lambda i: (0, 0)),
        ],
        out_specs=pl.BlockSpec((NB, OUT), lambda i: (i, 0)),
        out_shape=jax.ShapeDtypeStruct((NPAD, OUT), jnp.float32),
    )(parts, l1w, l1b, fw, fb)


def _gat_sc(haug, sd, aev, ids4):
    mesh = plsc.VectorSubcoreMesh(core_axis_name="c", subcore_axis_name="s")

    @functools.partial(
        pl.kernel,
        out_type=jax.ShapeDtypeStruct((NC, NPAD, WA), jnp.float32),
        mesh=mesh,
        scratch_types=[
            pltpu.VMEM_SHARED((ACCN, WA), jnp.float32),
            pltpu.VMEM((N,), jnp.float32),
            pltpu.VMEM((N,), jnp.float32),
            pltpu.VMEM((16,), jnp.float32),
            pltpu.VMEM((3, 3, K), jnp.int32),
            pltpu.VMEM((2, K, WA), jnp.float32),
            pltpu.SemaphoreType.DMA,
            pltpu.SemaphoreType.DMA((2,)),
        ],
        compiler_params=pltpu.CompilerParams(
            needs_layout_passes=False, use_tc_tiling_on_sc=False),
    )
    def k(haug_h, sd_h, aev_h, ids_h, parts_h,
          acc, s_tab, d_tab, ae_tab, idsb, rows, isem, gsem):
        cid = lax.axis_index("c")
        sid = lax.axis_index("s")
        cw = jnp.where(cid == 0, C0, C1)
        gw = jnp.where(cid == 0, sid * C0, NS * C0 + sid * C1)
        zero16 = jnp.zeros((16,), jnp.float32)

        def zrows(r, carry):
            for c in range(WA // 16):
                rows[0, r, pl.ds(c * 16, 16)] = zero16
            return carry

        lax.fori_loop(0, K, zrows, 0)

        def zacc(kk, carry):
            pltpu.sync_copy(rows.at[0],
                            acc.at[pl.ds(sid * STRIPE + kk * K, K)])
            return carry

        nfull = STRIPE // K
        rem = STRIPE - nfull * K
        lax.fori_loop(0, nfull, zacc, 0)
        if rem:
            pltpu.sync_copy(rows.at[0, pl.ds(0, rem)],
                            acc.at[pl.ds(sid * STRIPE + nfull * K, rem)])

        pltpu.sync_copy(sd_h.at[0, pl.ds(0, N)], s_tab)
        pltpu.sync_copy(sd_h.at[1, pl.ds(0, N)], d_tab)
        pltpu.sync_copy(aev_h, ae_tab)
        plsc.subcore_barrier()

        lanes = lax.iota(jnp.int32, 16)

        pltpu.sync_copy(ids_h.at[:, gw], idsb.at[0])
        pltpu.async_copy(ids_h.at[:, gw + 1], idsb.at[1], isem)
        pltpu.async_copy(haug_h.at[idsb.at[0, 0]], rows.at[0], gsem.at[0])

        def chunk(m, carry):
            p = lax.rem(m, 2)
            s0 = lax.rem(m, 3)
            s1 = lax.rem(m + 1, 3)
            s2 = lax.rem(m + 2, 3)

            @pl.when(m + 1 < cw)
            def _():
                pltpu.make_async_copy(ids_h.at[:, gw + m], idsb.at[s1],
                                      isem).wait()

            @pl.when(m + 2 < cw)
            def _():
                pltpu.async_copy(ids_h.at[:, gw + m + 2], idsb.at[s2], isem)

            @pl.when(m + 1 < cw)
            def _():
                pltpu.async_copy(haug_h.at[idsb.at[s1, 0]], rows.at[1 - p],
                                 gsem.at[1 - p])

            pltpu.make_async_copy(haug_h.at[idsb.at[s0, 0]], rows.at[p],
                                  gsem.at[p]).wait()
            base = (gw + m) * K
            for g in range(K // 16):
                si = idsb[s0, 0, pl.ds(g * 16, 16)]
                di = idsb[s0, 1, pl.ds(g * 16, 16)]
                ei = idsb[s0, 2, pl.ds(g * 16, 16)]
                lg = (plsc.load_gather(s_tab, [si])
                      + plsc.load_gather(d_tab, [di])
                      + plsc.load_gather(ae_tab, [ei]))
                lg = jnp.maximum(lg, lg * 0.2)
                ex = jnp.exp(lg)
                ex = jnp.where(base + g * 16 + lanes < E, ex, 0.0)
                for i in range(16):
                    r = g * 16 + i
                    w = ex[i]
                    for c in range(WA // 16):
                        rows[p, r, pl.ds(c * 16, 16)] = (
                            rows[p, r, pl.ds(c * 16, 16)] * w)
            pltpu.sync_copy(rows.at[p], acc.at[idsb.at[s0, 1]], add=True)
            return carry

        lax.fori_loop(0, cw, chunk, 0)
        plsc.subcore_barrier()
        pltpu.sync_copy(acc.at[pl.ds(sid * STRIPE, STRIPE)],
                        parts_h.at[cid, pl.ds(sid * STRIPE, STRIPE)])

    return k(haug, sd, aev, ids4)


def kernel(x, edge_index, edge_type, W1, a1_src, a1_dst, a1_edge,
           W2, a2_src, a2_dst, a2_edge, l1_w, l1_b, f_w, f_b):
    xp = jnp.zeros((NPAD, D), jnp.float32).at[:N].set(x)
    pad = EPAD - E
    ids4 = jnp.stack([
        jnp.pad(edge_index[0], (0, pad)),
        jnp.pad(edge_index[1], (0, pad)),
        jnp.pad(edge_type, (0, pad)),
    ]).reshape(3, TOT, K)
    A21 = jnp.zeros((8, D), jnp.float32).at[0].set(a1_src).at[1].set(a1_dst)
    A22 = jnp.zeros((8, H), jnp.float32).at[0].set(a2_src).at[1].set(a2_dst)
    ae1 = jnp.zeros((16,), jnp.float32).at[:T].set(a1_edge)
    ae2 = jnp.zeros((16,), jnp.float32).at[:T].set(a2_edge)

    haug1, sd1 = _embed(xp, W1, A21)
    parts1 = _gat_sc(haug1, sd1, ae1, ids4)
    haug2, sd2 = _mid(parts1, W2, A22)
    parts2 = _gat_sc(haug2, sd2, ae2, ids4)
    outp = _head(parts2, l1_w, l1_b.reshape(1, H), f_w, f_b.reshape(1, OUT))
    return outp[:N]


# speedup vs baseline: 1.0986x; 1.0986x over previous
import functools

import jax
import jax.numpy as jnp
from jax import lax
from jax.experimental import pallas as pl
from jax.experimental.pallas import tpu as pltpu
from jax.experimental.pallas import tpu_sc as plsc

N = 10000
E = 320000
D = 128
H = 128
T = 4
OUT = 64

NPAD = 10240
WA = 144
NB = 1024
NC = 2
NS = 16
NW = NC * NS
K = 64
CPW = (E + NW * K - 1) // (NW * K)
C0 = 177
C1 = 2 * CPW - C0
TOT = NS * (C0 + C1)
EPAD = TOT * K
GRID = NPAD // NB
ACCN = N
STRIPE = ACCN // NS


def _embed_body(x_ref, w_ref, a2_ref, haug_ref, sd_ref):
    h = lax.dot_general(x_ref[...], w_ref[...], (((1,), (0,)), ((), ())),
                        preferred_element_type=jnp.float32,
                        precision=lax.Precision.HIGHEST)
    ones_col = (lax.broadcasted_iota(jnp.int32, (NB, WA - D), 1) == 0)
    haug_ref[...] = jnp.concatenate([h, ones_col.astype(jnp.float32)], axis=1)
    sd_ref[...] = lax.dot_general(a2_ref[...], h, (((1,), (1,)), ((), ())),
                                  preferred_element_type=jnp.float32,
                                  precision=lax.Precision.HIGHEST)


def _embed(xp, W, A2):
    return pl.pallas_call(
        _embed_body,
        grid=(GRID,),
        in_specs=[
            pl.BlockSpec((NB, D), lambda i: (i, 0)),
            pl.BlockSpec((D, H), lambda i: (0, 0)),
            pl.BlockSpec((8, D), lambda i: (0, 0)),
        ],
        out_specs=[
            pl.BlockSpec((NB, WA), lambda i: (i, 0)),
            pl.BlockSpec((8, NB), lambda i: (0, i)),
        ],
        out_shape=[
            jax.ShapeDtypeStruct((NPAD, WA), jnp.float32),
            jax.ShapeDtypeStruct((8, NPAD), jnp.float32),
        ],
    )(xp, W, A2)


def _mid_body(parts_ref, w_ref, a2_ref, haug_ref, sd_ref):
    p = parts_ref[0] + parts_ref[1]
    g = jnp.maximum(p[:, :D] / (p[:, D:D + 1] + 1e-16), 0.0)
    h = lax.dot_general(g, w_ref[...], (((1,), (0,)), ((), ())),
                        preferred_element_type=jnp.float32,
                        precision=lax.Precision.HIGHEST)
    ones_col = (lax.broadcasted_iota(jnp.int32, (NB, WA - D), 1) == 0)
    haug_ref[...] = jnp.concatenate([h, ones_col.astype(jnp.float32)], axis=1)
    sd_ref[...] = lax.dot_general(a2_ref[...], h, (((1,), (1,)), ((), ())),
                                  preferred_element_type=jnp.float32,
                                  precision=lax.Precision.HIGHEST)


def _mid(parts, W, A2):
    return pl.pallas_call(
        _mid_body,
        grid=(GRID,),
        in_specs=[
            pl.BlockSpec((NC, NB, WA), lambda i: (0, i, 0)),
            pl.BlockSpec((H, H), lambda i: (0, 0)),
            pl.BlockSpec((8, H), lambda i: (0, 0)),
        ],
        out_specs=[
            pl.BlockSpec((NB, WA), lambda i: (i, 0)),
            pl.BlockSpec((8, NB), lambda i: (0, i)),
        ],
        out_shape=[
            jax.ShapeDtypeStruct((NPAD, WA), jnp.float32),
            jax.ShapeDtypeStruct((8, NPAD), jnp.float32),
        ],
    )(parts, W, A2)


def _head_body(parts_ref, l1w_ref, l1b_ref, fw_ref, fb_ref, out_ref):
    p = parts_ref[0] + parts_ref[1]
    g = jnp.maximum(p[:, :D] / (p[:, D:D + 1] + 1e-16), 0.0)
    t = lax.dot_general(g, l1w_ref[...], (((1,), (0,)), ((), ())),
                        preferred_element_type=jnp.float32,
                        precision=lax.Precision.HIGHEST)
    t = jnp.maximum(t + l1b_ref[...], 0.0)
    o = lax.dot_general(t, fw_ref[...], (((1,), (0,)), ((), ())),
                        preferred_element_type=jnp.float32,
                        precision=lax.Precision.HIGHEST)
    out_ref[...] = o + fb_ref[...]


def _head(parts, l1w, l1b, fw, fb):
    return pl.pallas_call(
        _head_body,
        grid=(GRID,),
        in_specs=[
            pl.BlockSpec((NC, NB, WA), lambda i: (0, i, 0)),
            pl.BlockSpec((H, H), lambda i: (0, 0)),
            pl.BlockSpec((1, H), lambda i: (0, 0)),
            pl.BlockSpec((H, OUT), lambda i: (0, 0)),
            pl.BlockSpec((1, OUT), lambda i: (0, 0)),
        ],
        out_specs=pl.BlockSpec((NB, OUT), lambda i: (i, 0)),
        out_shape=jax.ShapeDtypeStruct((NPAD, OUT), jnp.float32),
    )(parts, l1w, l1b, fw, fb)


def _gat_sc(haug, sd, aev, ids4):
    mesh = plsc.VectorSubcoreMesh(core_axis_name="c", subcore_axis_name="s")

    @functools.partial(
        pl.kernel,
        out_type=jax.ShapeDtypeStruct((NC, NPAD, WA), jnp.float32),
        mesh=mesh,
        scratch_types=[
            pltpu.VMEM_SHARED((ACCN, WA), jnp.float32),
            pltpu.VMEM((N,), jnp.float32),
            pltpu.VMEM((N,), jnp.float32),
            pltpu.VMEM((16,), jnp.float32),
            pltpu.VMEM((3, 3, K), jnp.int32),
            pltpu.VMEM((2, K, WA), jnp.float32),
            pltpu.SemaphoreType.DMA,
            pltpu.SemaphoreType.DMA((2,)),
        ],
        compiler_params=pltpu.CompilerParams(
            needs_layout_passes=False, use_tc_tiling_on_sc=False),
    )
    def k(haug_h, sd_h, aev_h, ids_h, parts_h,
          acc, s_tab, d_tab, ae_tab, idsb, rows, isem, gsem):
        cid = lax.axis_index("c")
        sid = lax.axis_index("s")
        cw = jnp.where(cid == 0, C0, C1)
        gw = jnp.where(cid == 0, sid * C0, NS * C0 + sid * C1)
        zero16 = jnp.zeros((16,), jnp.float32)

        def zrows(r, carry):
            for c in range(WA // 16):
                rows[0, r, pl.ds(c * 16, 16)] = zero16
            return carry

        lax.fori_loop(0, K, zrows, 0)

        def zacc(kk, carry):
            pltpu.sync_copy(rows.at[0],
                            acc.at[pl.ds(sid * STRIPE + kk * K, K)])
            return carry

        nfull = STRIPE // K
        rem = STRIPE - nfull * K
        lax.fori_loop(0, nfull, zacc, 0)
        if rem:
            pltpu.sync_copy(rows.at[0, pl.ds(0, rem)],
                            acc.at[pl.ds(sid * STRIPE + nfull * K, rem)])

        pltpu.sync_copy(sd_h.at[0, pl.ds(0, N)], s_tab)
        pltpu.sync_copy(sd_h.at[1, pl.ds(0, N)], d_tab)
        pltpu.sync_copy(aev_h, ae_tab)
        plsc.subcore_barrier()

        lanes = lax.iota(jnp.int32, 16)

        pltpu.sync_copy(ids_h.at[:, gw], idsb.at[0])
        pltpu.async_copy(ids_h.at[:, gw + 1], idsb.at[1], isem)
        pltpu.async_copy(haug_h.at[idsb.at[0, 0]], rows.at[0], gsem.at[0])

        def chunk(m, carry):
            p = lax.rem(m, 2)
            s0 = lax.rem(m, 3)
            s1 = lax.rem(m + 1, 3)
            s2 = lax.rem(m + 2, 3)

            @pl.when(m + 1 < cw)
            def _():
                pltpu.make_async_copy(ids_h.at[:, gw + m], idsb.at[s1],
                                      isem).wait()

            @pl.when(m + 2 < cw)
            def _():
                pltpu.async_copy(ids_h.at[:, gw + m + 2], idsb.at[s2], isem)

            @pl.when(m + 1 < cw)
            def _():
                pltpu.async_copy(haug_h.at[idsb.at[s1, 0]], rows.at[1 - p],
                                 gsem.at[1 - p])

            pltpu.make_async_copy(haug_h.at[idsb.at[s0, 0]], rows.at[p],
                                  gsem.at[p]).wait()
            base = (gw + m) * K
            for g in range(K // 16):
                si = idsb[s0, 0, pl.ds(g * 16, 16)]
                di = idsb[s0, 1, pl.ds(g * 16, 16)]
                ei = idsb[s0, 2, pl.ds(g * 16, 16)]
                lg = (plsc.load_gather(s_tab, [si])
                      + plsc.load_gather(d_tab, [di])
                      + plsc.load_gather(ae_tab, [ei]))
                lg = jnp.maximum(lg, lg * 0.2)
                ex = jnp.exp(lg)
                ex = jnp.where(base + g * 16 + lanes < E, ex, 0.0)
                for i in range(16):
                    r = g * 16 + i
                    w = ex[i]
                    for c in range(WA // 16):
                        rows[p, r, pl.ds(c * 16, 16)] = (
                            rows[p, r, pl.ds(c * 16, 16)] * w)
            pltpu.sync_copy(rows.at[p], acc.at[idsb.at[s0, 1]], add=True)
            return carry

        lax.fori_loop(0, cw, chunk, 0)
        plsc.subcore_barrier()
        pltpu.sync_copy(acc.at[pl.ds(sid * STRIPE, STRIPE)],
                        parts_h.at[cid, pl.ds(sid * STRIPE, STRIPE)])

    return k(haug, sd, aev, ids4)


def kernel(x, edge_index, edge_type, W1, a1_src, a1_dst, a1_edge,
           W2, a2_src, a2_dst, a2_edge, l1_w, l1_b, f_w, f_b):
    xp = jnp.zeros((NPAD, D), jnp.float32).at[:N].set(x)
    pad = EPAD - E
    ids4 = jnp.stack([
        jnp.pad(edge_index[0], (0, pad)),
        jnp.pad(edge_index[1], (0, pad)),
        jnp.pad(edge_type, (0, pad)),
    ]).reshape(3, TOT, K)
    A21 = jnp.zeros((8, D), jnp.float32).at[0].set(a1_src).at[1].set(a1_dst)
    A22 = jnp.zeros((8, H), jnp.float32).at[0].set(a2_src).at[1].set(a2_dst)
    ae1 = jnp.zeros((16,), jnp.float32).at[:T].set(a1_edge)
    ae2 = jnp.zeros((16,), jnp.float32).at[:T].set(a2_edge)

    haug1, sd1 = _embed(xp, W1, A21)
    parts1 = _gat_sc(haug1, sd1, ae1, ids4)
    haug2, sd2 = _mid(parts1, W2, A22)
    parts2 = _gat_sc(haug2, sd2, ae2, ids4)
    outp = _head(parts2, l1_w, l1_b.reshape(1, H), f_w, f_b.reshape(1, OUT))
    return outp[:N]
